# baseline (speedup 1.0000x reference)
import functools

import jax
import jax.numpy as jnp
from jax.experimental import pallas as pl
from jax.experimental.pallas import tpu as pltpu

NHEADS = 3
LEAK = 0.1

_EBLK = 4000


def _mlp_body(z_ref, w1, b1, w2, b2, w3, b3, w4, b4, w5, b5, o_ref):
    z = z_ref[...]
    z = jax.nn.leaky_relu(z @ w1[...] + b1[...], LEAK)
    z = jax.nn.leaky_relu(z @ w2[...] + b2[...], LEAK)
    z = jax.nn.leaky_relu(z @ w3[...] + b3[...], LEAK)
    z = jax.nn.leaky_relu(z @ w4[...] + b4[...], LEAK)
    o_ref[...] = z @ w5[...] + b5[...]


def _edge_mlp(z, We1, be1, We2, be2, We3, be3, We4, be4, We5, be5):
    E = z.shape[0]
    nblk = E // _EBLK
    grid = (nblk,)
    full = lambda a: pl.BlockSpec(a.shape, lambda i: (0,) * a.ndim)
    return pl.pallas_call(
        _mlp_body,
        grid=grid,
        in_specs=[pl.BlockSpec((_EBLK, z.shape[1]), lambda i: (i, 0))]
        + [full(w) for w in (We1, be1, We2, be2, We3, be3, We4, be4, We5, be5)],
        out_specs=pl.BlockSpec((_EBLK, 2), lambda i: (i, 0)),
        out_shape=jax.ShapeDtypeStruct((E, 2), jnp.float32),
    )(z, We1, be1, We2, be2, We3, be3, We4, be4, We5, be5)


def _bn(x, g, b):
    m = x.mean(axis=0)
    v = x.var(axis=0)
    return (x - m) / jnp.sqrt(v + 1e-5) * g + b


def _gat(x, src, dst, W, a_s, a_d, b, H, F, N):
    h = (x @ W).reshape(N, H, F)
    al_s = (h * a_s[None, :, :]).sum(-1)
    al_d = (h * a_d[None, :, :]).sum(-1)
    logit = jax.nn.leaky_relu(al_s[src] + al_d[dst], 0.2)
    m = jax.ops.segment_max(logit, dst, num_segments=N)
    ex = jnp.exp(logit - m[dst])
    den = jax.ops.segment_sum(ex, dst, num_segments=N)
    coef = ex / den[dst]
    out = jax.ops.segment_sum(h[src] * coef[:, :, None], dst, num_segments=N)
    return out.mean(axis=1) + b


def kernel(x, edge_index, e, xbatch, bn_node_g, bn_node_b, bn_edge_g, bn_edge_b,
           W1, a_s1, a_d1, b1, W2, a_s2, a_d2, b2, W3, a_s3, a_d3, b3,
           We1, be1, We2, be2, We3, be3, We4, be4, We5, be5):
    N = x.shape[0]
    xb = _bn(x, bn_node_g, bn_node_b)
    eb = _bn(e, bn_edge_g, bn_edge_b)
    src0, dst0 = edge_index[0], edge_index[1]
    loop = jnp.arange(N, dtype=src0.dtype)
    src = jnp.concatenate([src0, loop])
    dst = jnp.concatenate([dst0, loop])
    h = _gat(xb, src, dst, W1, a_s1, a_d1, b1, NHEADS, 16, N)
    h = _gat(h, src, dst, W2, a_s2, a_d2, b2, NHEADS, 32, N)
    h = _gat(h, src, dst, W3, a_s3, a_d3, b3, NHEADS, 64, N)
    z = jnp.concatenate([h[src0], h[dst0], eb], axis=1)
    return _edge_mlp(z, We1, be1, We2, be2, We3, be3, We4, be4, We5, be5)


# speedup vs baseline: 18.5726x; 18.5726x over previous
import jax
import jax.numpy as jnp
from jax import lax
from jax.experimental import pallas as pl
from jax.experimental.pallas import tpu as pltpu
from jax.experimental.pallas import tpu_sc as plsc

N = 50000
E = 800000
H = 3
LEAK = 0.1
ALEAK = 0.2
EPS = 1e-5

B = 128
NBLK = E // B
NC, NS, NT = 2, 16, 32
RPT = N // NS
ZR = 125

_MESH = plsc.VectorSubcoreMesh(core_axis_name="c", subcore_axis_name="s",
                               num_cores=NC, num_subcores=NS)
_SC_PARAMS = pltpu.CompilerParams(needs_layout_passes=False,
                                  use_tc_tiling_on_sc=False)


def _s16(v):
    return jnp.full((16,), v, jnp.int32)


def _leaky(x, slope):
    return jnp.maximum(x, slope * x)


def _ex_body(src_h, dst_h, al_h, ex_h, srcv, dstv, sv, dv, exv, sem0, sem1):
    cid = lax.axis_index("c")
    sid = lax.axis_index("s")
    w = sid * NC + cid
    iota = lax.iota(jnp.int32, 16)
    per = NBLK // NT
    nblk = jnp.where(w < NBLK - per * NT, per + 1, per)

    def blk(i, _):
        base = (w + i * NT) * B
        pltpu.sync_copy(src_h.at[pl.ds(base, B)], srcv)
        pltpu.sync_copy(dst_h.at[pl.ds(base, B)], dstv)
        c0 = pltpu.async_copy(al_h.at[srcv], sv, sem0)
        c1 = pltpu.async_copy(al_h.at[dstv], dv, sem1)
        c0.wait()
        c1.wait()
        for g in range(B // 16):
            rows = g * 16 + iota
            for h in range(H):
                a = plsc.load_gather(sv, [rows, _s16(h)])
                b = plsc.load_gather(dv, [rows, _s16(3 + h)])
                lo = a + b
                plsc.store_scatter(exv, [rows, _s16(h)],
                                   jnp.exp(_leaky(lo, ALEAK)))
        pltpu.sync_copy(exv, ex_h.at[pl.ds(base, B)])
        return 0

    lax.fori_loop(0, nblk, blk, 0)


def _ex_pass(src, dst, al):
    return pl.kernel(
        _ex_body,
        out_type=jax.ShapeDtypeStruct((E, 4), jnp.float32),
        mesh=_MESH,
        scratch_types=[
            pltpu.VMEM((B,), jnp.int32),
            pltpu.VMEM((B,), jnp.int32),
            pltpu.VMEM((B, 16), jnp.float32),
            pltpu.VMEM((B, 16), jnp.float32),
            pltpu.VMEM((B, 4), jnp.float32),
            pltpu.SemaphoreType.DMA,
            pltpu.SemaphoreType.DMA,
        ],
        compiler_params=_SC_PARAMS,
    )(src, dst, al)


SCHED1 = {
    0: [[("num", 0, 0), ("den", 0)], [("num", 1, 1), ("den", 1)]],
    1: [[("num", 2, 2), ("den", 2)], []],
}
SCHED2 = {
    0: [[("num", 0, 0), ("num", 0, 1)], [("num", 1, 2), ("num", 1, 3)],
        [("den", 2)]],
    1: [[("num", 2, 4), ("num", 2, 5)], [("den", 0), ("den", 1)], []],
}
SCHED3 = {
    0: [[("num", 0, 0), ("num", 0, 1)], [("num", 0, 2), ("num", 0, 3)],
        [("num", 1, 4), ("num", 1, 5)], [("num", 1, 6), ("num", 1, 7)]],
    1: [[("num", 2, 8), ("num", 2, 9)], [("num", 2, 10), ("num", 2, 11)],
        [("den", 0), ("den", 1)], [("den", 2)]],
}


def _sched_meta(sched, F):
    num_src = {}
    den_src = {}
    pos = 0
    for c in (0, 1):
        for chunks in sched[c]:
            if not chunks:
                continue
            for k, ch in enumerate(chunks):
                if ch[0] == "num":
                    fs = ch[2] - ch[1] * (F // 16)
                    num_src[(ch[1], fs)] = (pos, 16 * k)
                else:
                    den_src[ch[1]] = (pos, 16 * k)
            pos += 1
    return num_src, den_src, pos


def _make_num_pass(sched, n_h):
    n_out = _sched_meta(sched, 16)[2]

    def body(*refs):
        (src_h, dst_h, ex_h), refs = refs[:3], refs[3:]
        h_arrs, refs = refs[:n_h], refs[n_h:]
        (zin_h,), refs = refs[:1], refs[1:]
        outs, refs = refs[:n_out], refs[n_out:]
        srcv, dstv, exv, g0, g1, scl, zbuf, T, sem0, sem1 = refs

        cid = lax.axis_index("c")
        sid = lax.axis_index("s")
        iota = lax.iota(jnp.int32, 16)
        per = NBLK // NS
        nblk = jnp.where(sid < NBLK - per * NS, per + 1, per)

        pltpu.sync_copy(zin_h, zbuf)

        out_pos = {}
        pos = 0
        for c in (0, 1):
            for r, chunks in enumerate(sched[c]):
                if chunks:
                    out_pos[(c, r)] = pos
                    pos += 1

        max_r = max(len(sched[0]), len(sched[1]))
        gbufs = [g0, g1]
        for r in range(max_r):
            for c in (0, 1):
                chunks = sched[c][r] if r < len(sched[c]) else []
                if not chunks:
                    continue

                @pl.when(cid == c)
                def _zero():
                    def zb(j, _):
                        pltpu.sync_copy(
                            zbuf, T.at[pl.ds(sid * RPT + j * ZR, ZR)])
                        return 0
                    lax.fori_loop(0, RPT // ZR, zb, 0)

            plsc.subcore_barrier()

            for c in (0, 1):
                chunks = sched[c][r] if r < len(sched[c]) else []
                if not chunks:
                    continue

                @pl.when(cid == c)
                def _edges(chunks=chunks):
                    has_num = any(ch[0] == "num" for ch in chunks)

                    def blk(i, _):
                        base = (sid + i * NS) * B
                        pltpu.sync_copy(dst_h.at[pl.ds(base, B)], dstv)
                        pltpu.sync_copy(ex_h.at[pl.ds(base, B)], exv)
                        if has_num:
                            pltpu.sync_copy(src_h.at[pl.ds(base, B)], srcv)
                        waits = []
                        for k, ch in enumerate(chunks):
                            if ch[0] == "num":
                                waits.append(pltpu.async_copy(
                                    h_arrs[ch[2]].at[srcv], gbufs[k],
                                    [sem0, sem1][k]))
                        for wd in waits:
                            wd.wait()

                        def eb(ii, _):
                            for j in range(4):
                                b = ii * 4 + j
                                for k, ch in enumerate(chunks):
                                    spl = plsc.load_gather(
                                        exv, [_s16(b), _s16(ch[1])])
                                    if ch[0] == "num":
                                        row = plsc.load_gather(
                                            gbufs[k], [_s16(b), iota])
                                        val = row * spl
                                    else:
                                        val = spl
                                    plsc.store_scatter(
                                        scl, [_s16(b), iota + 16 * k], val)
                            return 0

                        lax.fori_loop(0, B // 4, eb, 0)
                        pltpu.sync_copy(scl, T.at[dstv], add=True)
                        return 0

                    lax.fori_loop(0, nblk, blk, 0)

            plsc.subcore_barrier()

            for c in (0, 1):
                chunks = sched[c][r] if r < len(sched[c]) else []
                if not chunks:
                    continue

                @pl.when(cid == c)
                def _dump(p=out_pos[(c, r)]):
                    pltpu.sync_copy(
                        T.at[pl.ds(sid * RPT, RPT)],
                        outs[p].at[pl.ds(sid * RPT, RPT)])

    def run(src, dst, ex, h_arrs, zin):
        return pl.kernel(
            body,
            out_type=[jax.ShapeDtypeStruct((N, 32), jnp.float32)] * n_out,
            mesh=_MESH,
            scratch_types=[
                pltpu.VMEM((B,), jnp.int32),
                pltpu.VMEM((B,), jnp.int32),
                pltpu.VMEM((B, 4), jnp.float32),
                pltpu.VMEM((B, 16), jnp.float32),
                pltpu.VMEM((B, 16), jnp.float32),
                pltpu.VMEM((B, 32), jnp.float32),
                pltpu.VMEM((ZR, 32), jnp.float32),
                pltpu.VMEM_SHARED((N, 32), jnp.float32),
                pltpu.SemaphoreType.DMA,
                pltpu.SemaphoreType.DMA,
            ],
            compiler_params=_SC_PARAMS,
        )(src, dst, ex, *h_arrs, zin)

    return run


_num_pass1 = _make_num_pass(SCHED1, 3)
_num_pass2 = _make_num_pass(SCHED2, 6)
_num_pass3 = _make_num_pass(SCHED3, 12)


def _pq_body(src_h, dst_h, p_h, q_h, sp_h, sq_h, srcv, dstv, bp, bq,
             sem0, sem1):
    cid = lax.axis_index("c")
    sid = lax.axis_index("s")
    w = sid * NC + cid
    per = NBLK // NT
    nblk = jnp.where(w < NBLK - per * NT, per + 1, per)

    def blk(i, _):
        base = (w + i * NT) * B
        pltpu.sync_copy(src_h.at[pl.ds(base, B)], srcv)
        pltpu.sync_copy(dst_h.at[pl.ds(base, B)], dstv)
        c0 = pltpu.async_copy(p_h.at[srcv], bp, sem0)
        c1 = pltpu.async_copy(q_h.at[dstv], bq, sem1)
        c0.wait()
        pltpu.sync_copy(bp, sp_h.at[pl.ds(base, B)])
        c1.wait()
        pltpu.sync_copy(bq, sq_h.at[pl.ds(base, B)])
        return 0

    lax.fori_loop(0, nblk, blk, 0)


def _pq_pass(src, dst, p, q):
    return pl.kernel(
        _pq_body,
        out_type=[jax.ShapeDtypeStruct((E, 64), jnp.float32)] * 2,
        mesh=_MESH,
        scratch_types=[
            pltpu.VMEM((B,), jnp.int32),
            pltpu.VMEM((B,), jnp.int32),
            pltpu.VMEM((B, 64), jnp.float32),
            pltpu.VMEM((B, 64), jnp.float32),
            pltpu.SemaphoreType.DMA,
            pltpu.SemaphoreType.DMA,
        ],
        compiler_params=_SC_PARAMS,
    )(src, dst, p, q)


def _xstats_body(x_ref, s_ref, q_ref):
    i = pl.program_id(0)
    z = x_ref[...]
    s = jnp.broadcast_to(jnp.sum(z, axis=0, keepdims=True), (8, 16))
    q = jnp.broadcast_to(jnp.sum(z * z, axis=0, keepdims=True), (8, 16))

    @pl.when(i == 0)
    def _():
        s_ref[...] = s
        q_ref[...] = q

    @pl.when(i > 0)
    def _():
        s_ref[...] += s
        q_ref[...] += q


def _xstats(x):
    XB = 5000
    return pl.pallas_call(
        _xstats_body,
        grid=(N // XB,),
        in_specs=[pl.BlockSpec((XB, 16), lambda i: (i, 0))],
        out_specs=[pl.BlockSpec((8, 16), lambda i: (0, 0))] * 2,
        out_shape=[jax.ShapeDtypeStruct((8, 16), jnp.float32)] * 2,
    )(x)


def _k1_body(x_ref, xs_ref, xq_ref, g_ref, b_ref, w_ref, as_ref,
             h0, h1, h2, al_ref):
    x = x_ref[...]
    m = xs_ref[0:1, :] * (1.0 / N)
    v = xq_ref[0:1, :] * (1.0 / N) - m * m
    xb = (x - m) * (g_ref[...] / jnp.sqrt(v + EPS)) + b_ref[...]
    hf = jnp.dot(xb, w_ref[...], preferred_element_type=jnp.float32)
    h0[...] = hf[:, 0:16]
    h1[...] = hf[:, 16:32]
    h2[...] = hf[:, 32:48]
    al_ref[...] = jnp.dot(hf, as_ref[...], preferred_element_type=jnp.float32)


def _k1(x, g, b, W1, as1):
    XB = 5000
    xs, xq = _xstats(x)
    full = lambda a: pl.BlockSpec(a.shape, lambda i: (0,) * a.ndim)
    args = [x, xs, xq, g.reshape(1, 16), b.reshape(1, 16), W1, as1]
    return pl.pallas_call(
        _k1_body,
        grid=(N // XB,),
        in_specs=[pl.BlockSpec((XB, 16), lambda i: (i, 0))]
        + [full(a) for a in args[1:]],
        out_specs=[pl.BlockSpec((XB, 16), lambda i: (i, 0))] * 4,
        out_shape=[jax.ShapeDtypeStruct((N, 16), jnp.float32)] * 4,
    )(*args)


def _make_k2(sched, F, n_h, FN):
    num_src, den_src, n_out = _sched_meta(sched, F)
    nfs = F // 16

    def body(rs, al_ref, h_arrs, b_ref, wn_ref, asn_ref, outs):
        als = al_ref[...]
        acc = None
        for h in range(H):
            exs = jnp.exp(_leaky(als[:, h:h + 1] + als[:, 3 + h:4 + h], ALEAK))
            num = jnp.concatenate(
                [rs[num_src[(h, fs)][0]]
                 [:, num_src[(h, fs)][1]:num_src[(h, fs)][1] + 16]
                 for fs in range(nfs)], axis=1)
            hself = jnp.concatenate(
                [h_arrs[h * nfs + fs][...] for fs in range(nfs)], axis=1)
            num = num + exs * hself
            dpos, doff = den_src[h]
            den = rs[dpos][:, doff:doff + 1] + exs
            term = num / den
            acc = term if acc is None else acc + term
        out = acc * (1.0 / H) + b_ref[...]
        hn = jnp.dot(out, wn_ref[...], preferred_element_type=jnp.float32)
        if asn_ref is not None:
            for j in range(FN // 16):
                outs[j][...] = hn[:, 16 * j:16 * j + 16]
            outs[FN // 16][...] = jnp.dot(
                hn, asn_ref[...], preferred_element_type=jnp.float32)
        else:
            outs[0][...] = hn[:, 0:64]
            outs[1][...] = hn[:, 64:128]

    RB = 1000
    grid = (N // RB,)

    def run(rs, al, h_arrs, bias, wn, asn):
        has_asn = asn is not None
        if has_asn:
            out_shape = ([jax.ShapeDtypeStruct((N, 16), jnp.float32)]
                         * (FN // 16)
                         + [jax.ShapeDtypeStruct((N, 16), jnp.float32)])
            out_specs = [pl.BlockSpec((RB, 16), lambda i: (i, 0))
                         for _ in range(FN // 16 + 1)]
        else:
            out_shape = [jax.ShapeDtypeStruct((N, 64), jnp.float32)] * 2
            out_specs = [pl.BlockSpec((RB, 64), lambda i: (i, 0))] * 2

        row_spec16 = pl.BlockSpec((RB, 16), lambda i: (i, 0))
        row_spec32 = pl.BlockSpec((RB, 32), lambda i: (i, 0))
        full = lambda a: pl.BlockSpec(a.shape, lambda i: (0,) * a.ndim)

        args = list(rs) + [al] + list(h_arrs) + [bias, wn]
        in_specs = ([row_spec32] * len(rs) + [row_spec16]
                    + [row_spec16] * len(h_arrs) + [full(bias), full(wn)])
        if has_asn:
            args.append(asn)
            in_specs.append(full(asn))

        def body_call(*refs):
            ins = list(refs[:len(args)])
            outs2 = refs[len(args):]
            asn_ref = ins.pop() if has_asn else None
            wn_r = ins.pop()
            b_r = ins.pop()
            body(ins[:len(rs)], ins[len(rs)], ins[len(rs) + 1:],
                 b_r, wn_r, asn_ref, outs2)

        return pl.pallas_call(
            body_call,
            grid=grid,
            in_specs=in_specs,
            out_specs=out_specs,
            out_shape=out_shape,
        )(*args)

    return run


_k2_1 = _make_k2(SCHED1, 16, 3, 96)
_k2_2 = _make_k2(SCHED2, 32, 6, 192)
_k2_3 = _make_k2(SCHED3, 64, 12, 128)


def _estats_body(e_ref, s_ref, q_ref):
    i = pl.program_id(0)
    z = e_ref[...]
    s = jnp.broadcast_to(jnp.sum(z, axis=0, keepdims=True), (8, 10))
    q = jnp.broadcast_to(jnp.sum(z * z, axis=0, keepdims=True), (8, 10))

    @pl.when(i == 0)
    def _():
        s_ref[...] = s
        q_ref[...] = q

    @pl.when(i > 0)
    def _():
        s_ref[...] += s
        q_ref[...] += q


def _estats(e):
    EB = 8000
    return pl.pallas_call(
        _estats_body,
        grid=(E // EB,),
        in_specs=[pl.BlockSpec((EB, 10), lambda i: (i, 0))],
        out_specs=[pl.BlockSpec((8, 10), lambda i: (0, 0))] * 2,
        out_shape=[jax.ShapeDtypeStruct((8, 10), jnp.float32)] * 2,
    )(e)


def _mlp_body(sp, sq, e_ref, es_ref, eq_ref, g_ref, bb_ref, c_ref, be1,
              w2, b2, w3, b3, w4, b4, w5, b5, o_ref):
    m = es_ref[0:1, :] * (1.0 / E)
    v = eq_ref[0:1, :] * (1.0 / E) - m * m
    sc = g_ref[...] / jnp.sqrt(v + EPS)
    eb = (e_ref[...] - m) * sc + bb_ref[...]
    z = sp[...] + sq[...] + jnp.dot(
        eb, c_ref[...], preferred_element_type=jnp.float32) + be1[...]
    z = _leaky(z, LEAK)
    z = _leaky(jnp.dot(z, w2[...], preferred_element_type=jnp.float32)
               + b2[...], LEAK)
    z = _leaky(jnp.dot(z, w3[...], preferred_element_type=jnp.float32)
               + b3[...], LEAK)
    z = _leaky(jnp.dot(z, w4[...], preferred_element_type=jnp.float32)
               + b4[...], LEAK)
    o_ref[...] = jnp.dot(z, w5[...], preferred_element_type=jnp.float32) \
        + b5[...]


def _mlp(sp, sq, e, es, eq, g, bb, C, be1, w2, b2, w3, b3, w4, b4, w5, b5):
    EB = 4000
    full = lambda a: pl.BlockSpec(a.shape, lambda i: (0,) * a.ndim)
    row = lambda w: pl.BlockSpec((EB, w), lambda i: (i, 0))
    args = [sp, sq, e, es, eq, g, bb, C, be1, w2, b2, w3, b3, w4, b4, w5, b5]
    in_specs = [row(64), row(64), row(10)] + [full(a) for a in args[3:]]
    return pl.pallas_call(
        _mlp_body,
        grid=(E // EB,),
        in_specs=in_specs,
        out_specs=pl.BlockSpec((EB, 2), lambda i: (i, 0)),
        out_shape=jax.ShapeDtypeStruct((E, 2), jnp.float32),
    )(*args)


def _as_mat(a_s, a_d, F):
    M = jnp.zeros((H * F, 16), jnp.float32)
    rows = jnp.arange(H * F)
    heads = jnp.repeat(jnp.arange(H), F)
    M = M.at[rows, heads].set(a_s.reshape(-1))
    M = M.at[rows, 3 + heads].set(a_d.reshape(-1))
    return M


def kernel(x, edge_index, e, xbatch, bn_node_g, bn_node_b, bn_edge_g,
           bn_edge_b, W1, a_s1, a_d1, b1, W2, a_s2, a_d2, b2, W3, a_s3,
           a_d3, b3, We1, be1, We2, be2, We3, be3, We4, be4, We5, be5):
    src = edge_index[0]
    dst = edge_index[1]
    zin = jnp.zeros((ZR, 32), jnp.float32)

    as1 = _as_mat(a_s1, a_d1, 16)
    as2 = _as_mat(a_s2, a_d2, 32)
    as3 = _as_mat(a_s3, a_d3, 64)

    *h1_arrs, al1 = _k1(x, bn_node_g, bn_node_b, W1, as1)
    ex1 = _ex_pass(src, dst, al1)
    r1 = _num_pass1(src, dst, ex1, h1_arrs, zin)
    out2 = _k2_1(r1, al1, h1_arrs, b1.reshape(1, 16), W2, as2)
    h2_arrs, al2 = out2[:6], out2[6]

    ex2 = _ex_pass(src, dst, al2)
    r2 = _num_pass2(src, dst, ex2, h2_arrs, zin)
    out3 = _k2_2(r2, al2, h2_arrs, b2.reshape(1, 32), W3, as3)
    h3_arrs, al3 = out3[:12], out3[12]

    ex3 = _ex_pass(src, dst, al3)
    r3 = _num_pass3(src, dst, ex3, h3_arrs, zin)
    pq_w = jnp.concatenate([We1[0:64], We1[64:128]], axis=1)
    p, q = _k2_3(r3, al3, h3_arrs, b3.reshape(1, 64), pq_w, None)

    sp, sq = _pq_pass(src, dst, p, q)
    es, eq = _estats(e)
    return _mlp(sp, sq, e, es, eq, bn_edge_g.reshape(1, 10),
                bn_edge_b.reshape(1, 10), We1[128:138], be1.reshape(1, 64),
                We2, be2.reshape(1, 32), We3, be3.reshape(1, 16),
                We4, be4.reshape(1, 8), We5, be5.reshape(1, 2))


# speedup vs baseline: 19.8085x; 1.0665x over previous
import jax
import jax.numpy as jnp
from jax import lax
from jax.experimental import pallas as pl
from jax.experimental.pallas import tpu as pltpu
from jax.experimental.pallas import tpu_sc as plsc

N = 50000
E = 800000
H = 3
LEAK = 0.1
ALEAK = 0.2
EPS = 1e-5

B = 128
NBLK = E // B
NC, NS, NT = 2, 16, 32
RPT = N // NS
ZR = 125
HALF = NBLK // 2
NBPT = 196

_MESH = plsc.VectorSubcoreMesh(core_axis_name="c", subcore_axis_name="s",
                               num_cores=NC, num_subcores=NS)
_SC_PARAMS = pltpu.CompilerParams(needs_layout_passes=False,
                                  use_tc_tiling_on_sc=False)


def _s16(v):
    return jnp.full((16,), v, jnp.int32)


def _leaky(x, slope):
    return jnp.maximum(x, slope * x)


def _copy128(src, dst):
    for q in range(8):
        dst[pl.ds(q * 16, 16)] = src[pl.ds(q * 16, 16)]


def _ex_body(src_h, dst_h, al_h, ex_h, trash_h,
             srcv0, srcv1, dstv0, dstv1, sv0, sv1, dv0, dv1, exv0, exv1,
             si0, si1, sg0, sg1, sw0, sw1):
    srcv = [srcv0, srcv1]
    dstv = [dstv0, dstv1]
    sv = [sv0, sv1]
    dv = [dv0, dv1]
    exv = [exv0, exv1]
    si = [si0, si1]
    sg = [sg0, sg1]
    sw = [sw0, sw1]

    cid = lax.axis_index("c")
    sid = lax.axis_index("s")
    w = sid * NC + cid
    iota = lax.iota(jnp.int32, 16)
    per = NBLK // NT
    jmax = jnp.where(w < NBLK - per * NT, per, per - 1)

    def base_of(j):
        jj = jnp.minimum(j, jmax)
        return (w + jj * NT) * B

    def idx_start(j, p):
        base = base_of(j)
        pltpu.make_async_copy(src_h.at[pl.ds(base, B)], srcv[p], si[p]).start()
        pltpu.make_async_copy(dst_h.at[pl.ds(base, B)], dstv[p], si[p]).start()

    def idx_wait(p):
        pltpu.make_async_copy(src_h.at[pl.ds(0, B)], srcv[p], si[p]).wait()
        pltpu.make_async_copy(dst_h.at[pl.ds(0, B)], dstv[p], si[p]).wait()

    def gath_start(p):
        pltpu.make_async_copy(al_h.at[srcv[p]], sv[p], sg[p]).start()
        pltpu.make_async_copy(al_h.at[dstv[p]], dv[p], sg[p]).start()

    def gath_wait(p):
        pltpu.make_async_copy(al_h.at[srcv[p]], sv[p], sg[p]).wait()
        pltpu.make_async_copy(al_h.at[dstv[p]], dv[p], sg[p]).wait()

    idx_start(0, 0)
    idx_start(1, 1)
    idx_wait(0)
    gath_start(0)
    pltpu.make_async_copy(exv[0], trash_h, sw[0]).start()
    pltpu.make_async_copy(exv[1], trash_h, sw[1]).start()

    def pair(t, _):
        for p in (0, 1):
            j = 2 * t + p
            gath_wait(p)
            pltpu.make_async_copy(exv[p], trash_h, sw[p]).wait()

            def grp_body(g, _):
                rows = g * 16 + iota
                for h in range(H):
                    a = plsc.load_gather(sv[p], [rows, _s16(h)])
                    b = plsc.load_gather(dv[p], [rows, _s16(3 + h)])
                    lo = a + b
                    plsc.store_scatter(exv[p], [rows, _s16(h)],
                                       jnp.exp(_leaky(lo, ALEAK)))
                return 0

            lax.fori_loop(0, B // 16, grp_body, 0)
            base = base_of(j)
            pltpu.make_async_copy(exv[p], ex_h.at[pl.ds(base, B)],
                                  sw[p]).start()
            idx_start(j + 2, p)
            idx_wait(1 - p)
            gath_start(1 - p)
        return 0

    lax.fori_loop(0, NBPT // 2, pair, 0)
    gath_wait(0)
    idx_wait(1)
    pltpu.make_async_copy(exv[0], trash_h, sw[0]).wait()
    pltpu.make_async_copy(exv[1], trash_h, sw[1]).wait()


def _ex_pass(src, dst, al):
    ex, _ = pl.kernel(
        _ex_body,
        out_type=[jax.ShapeDtypeStruct((E, 4), jnp.float32),
                  jax.ShapeDtypeStruct((B, 4), jnp.float32)],
        mesh=_MESH,
        scratch_types=[
            pltpu.VMEM((B,), jnp.int32), pltpu.VMEM((B,), jnp.int32),
            pltpu.VMEM((B,), jnp.int32), pltpu.VMEM((B,), jnp.int32),
            pltpu.VMEM((B, 16), jnp.float32), pltpu.VMEM((B, 16), jnp.float32),
            pltpu.VMEM((B, 16), jnp.float32), pltpu.VMEM((B, 16), jnp.float32),
            pltpu.VMEM((B, 4), jnp.float32), pltpu.VMEM((B, 4), jnp.float32),
            pltpu.SemaphoreType.DMA, pltpu.SemaphoreType.DMA,
            pltpu.SemaphoreType.DMA, pltpu.SemaphoreType.DMA,
            pltpu.SemaphoreType.DMA, pltpu.SemaphoreType.DMA,
        ],
        compiler_params=_SC_PARAMS,
    )(src, dst, al)
    return ex


SCHED1 = [[("num", 0, 0), ("den", 0)],
          [("num", 1, 1), ("den", 1)],
          [("num", 2, 2), ("den", 2)]]
SCHED2 = [[("num", 0, 0), ("num", 0, 1)],
          [("num", 1, 2), ("num", 1, 3)],
          [("num", 2, 4), ("num", 2, 5)],
          [("den", 0), ("den", 1)],
          [("den", 2)]]
SCHED3 = [[("num", 0, 0), ("num", 0, 1)],
          [("num", 0, 2), ("num", 0, 3)],
          [("num", 1, 4), ("num", 1, 5)],
          [("num", 1, 6), ("num", 1, 7)],
          [("num", 2, 8), ("num", 2, 9)],
          [("num", 2, 10), ("num", 2, 11)],
          [("den", 0), ("den", 1)],
          [("den", 2)]]


def _sched_meta(sched, F):
    num_src = {}
    den_src = {}
    for r, chunks in enumerate(sched):
        for k, ch in enumerate(chunks):
            if ch[0] == "num":
                fs = ch[2] - ch[1] * (F // 16)
                num_src[(ch[1], fs)] = (r, 16 * k)
            else:
                den_src[ch[1]] = (r, 16 * k)
    return num_src, den_src, len(sched)


def _make_num_pass(sched, n_h):
    nrounds = len(sched)
    n_out = 2 * nrounds

    def body(*refs):
        (src_h, dst_h, ex_h), refs = refs[:3], refs[3:]
        h_arrs, refs = refs[:n_h], refs[n_h:]
        (zin_h,), refs = refs[:1], refs[1:]
        outs, refs = refs[:n_out], refs[n_out:]
        (srcv0, srcv1, dstv0, dstv1, dsc0, dsc1, exv0, exv1,
         ga0, ga1, gb0, gb1, scl0, scl1, zbuf, T,
         si0, si1, sg0, sg1, ss0, ss1) = refs
        srcv = [srcv0, srcv1]
        dstv = [dstv0, dstv1]
        dsc = [dsc0, dsc1]
        exv = [exv0, exv1]
        gk = [[ga0, gb0], [ga1, gb1]]
        scl = [scl0, scl1]
        si = [si0, si1]
        sg = [sg0, sg1]
        ss = [ss0, ss1]

        cid = lax.axis_index("c")
        sid = lax.axis_index("s")
        iota = lax.iota(jnp.int32, 16)
        chbase = cid * HALF
        per = HALF // NS
        jmax = jnp.where(sid < HALF - per * NS, per, per - 1)

        pltpu.sync_copy(zin_h, zbuf)

        for r, chunks in enumerate(sched):
            numk = [k for k, ch in enumerate(chunks) if ch[0] == "num"]

            def base_of(j):
                jj = jnp.minimum(j, jmax)
                return (chbase + sid + jj * NS) * B

            def idx_start(j, p):
                base = base_of(j)
                pltpu.make_async_copy(
                    dst_h.at[pl.ds(base, B)], dstv[p], si[p]).start()
                pltpu.make_async_copy(
                    ex_h.at[pl.ds(base, B)], exv[p], si[p]).start()
                if numk:
                    pltpu.make_async_copy(
                        src_h.at[pl.ds(base, B)], srcv[p], si[p]).start()

            def idx_wait(p):
                pltpu.make_async_copy(
                    dst_h.at[pl.ds(0, B)], dstv[p], si[p]).wait()
                pltpu.make_async_copy(
                    ex_h.at[pl.ds(0, B)], exv[p], si[p]).wait()
                if numk:
                    pltpu.make_async_copy(
                        src_h.at[pl.ds(0, B)], srcv[p], si[p]).wait()

            def gath_start(p):
                for k in numk:
                    pltpu.make_async_copy(
                        h_arrs[chunks[k][2]].at[srcv[p]], gk[p][k],
                        sg[p]).start()

            def gath_wait(p):
                for k in numk:
                    pltpu.make_async_copy(
                        h_arrs[chunks[k][2]].at[srcv[p]], gk[p][k],
                        sg[p]).wait()

            def scat_start(p):
                pltpu.make_async_copy(
                    scl[p], T.at[dsc[p]], ss[p]).start(add=True)

            def scat_wait(p):
                pltpu.make_async_copy(scl[p], T.at[dsc[p]], ss[p]).wait()

            def zb(jz, _):
                pltpu.sync_copy(zbuf, T.at[pl.ds(sid * RPT + jz * ZR, ZR)])
                return 0

            lax.fori_loop(0, RPT // ZR, zb, 0)
            plsc.subcore_barrier()

            idx_start(0, 0)
            idx_start(1, 1)
            idx_wait(0)
            _copy128(dstv[0], dsc[0])
            gath_start(0)

            def zscl(jz, _):
                z16 = jnp.zeros((16,), jnp.float32)
                for p in (0, 1):
                    plsc.store_scatter(
                        scl[p], [(jz * 16 + iota) // 32,
                                 (jz * 16 + iota) % 32], z16)
                return 0

            lax.fori_loop(0, B * 32 // 16, zscl, 0)
            _copy128(dstv[0], dsc[1])
            scat_start(0)
            scat_start(1)

            def pair(t, _):
                for p in (0, 1):
                    j = 2 * t + p
                    gath_wait(p)
                    vf = jnp.where(
                        (sid + j * NS) < HALF, 1.0, 0.0).astype(jnp.float32)

                    def grp_body(g, _):
                        rows = g * 16 + iota
                        for k, ch in enumerate(chunks):
                            exg = plsc.load_gather(
                                exv[p], [rows, _s16(ch[1])]) * vf
                            if ch[0] == "num":
                                def fb(fi, _, k=k, exg=exg, rows=rows):
                                    for f2 in range(8):
                                        f = fi * 8 + f2
                                        col = plsc.load_gather(
                                            gk[p][k], [rows, _s16(f)])
                                        plsc.store_scatter(
                                            scl[p], [rows, _s16(16 * k + f)],
                                            col * exg)
                                    return 0
                                lax.fori_loop(0, 2, fb, 0)
                            else:
                                def fb(fi, _, k=k, exg=exg, rows=rows):
                                    for f2 in range(8):
                                        f = fi * 8 + f2
                                        plsc.store_scatter(
                                            scl[p], [rows, _s16(16 * k + f)],
                                            exg)
                                    return 0
                                lax.fori_loop(0, 2, fb, 0)
                        return 0

                    lax.fori_loop(0, B // 16, grp_body, 0)
                    scat_wait(p)
                    _copy128(dstv[p], dsc[p])
                    scat_start(p)
                    idx_start(j + 2, p)
                    idx_wait(1 - p)
                    gath_start(1 - p)
                return 0

            lax.fori_loop(0, NBPT // 2, pair, 0)
            gath_wait(0)
            idx_wait(1)
            scat_wait(0)
            scat_wait(1)
            plsc.subcore_barrier()

            @pl.when(cid == 0)
            def _(r=r):
                pltpu.sync_copy(T.at[pl.ds(sid * RPT, RPT)],
                                outs[2 * r].at[pl.ds(sid * RPT, RPT)])

            @pl.when(cid == 1)
            def _(r=r):
                pltpu.sync_copy(T.at[pl.ds(sid * RPT, RPT)],
                                outs[2 * r + 1].at[pl.ds(sid * RPT, RPT)])

    def run(src, dst, ex, h_arrs, zin):
        return pl.kernel(
            body,
            out_type=[jax.ShapeDtypeStruct((N, 32), jnp.float32)] * n_out,
            mesh=_MESH,
            scratch_types=[
                pltpu.VMEM((B,), jnp.int32), pltpu.VMEM((B,), jnp.int32),
                pltpu.VMEM((B,), jnp.int32), pltpu.VMEM((B,), jnp.int32),
                pltpu.VMEM((B,), jnp.int32), pltpu.VMEM((B,), jnp.int32),
                pltpu.VMEM((B, 4), jnp.float32),
                pltpu.VMEM((B, 4), jnp.float32),
                pltpu.VMEM((B, 16), jnp.float32),
                pltpu.VMEM((B, 16), jnp.float32),
                pltpu.VMEM((B, 16), jnp.float32),
                pltpu.VMEM((B, 16), jnp.float32),
                pltpu.VMEM((B, 32), jnp.float32),
                pltpu.VMEM((B, 32), jnp.float32),
                pltpu.VMEM((ZR, 32), jnp.float32),
                pltpu.VMEM_SHARED((N, 32), jnp.float32),
                pltpu.SemaphoreType.DMA, pltpu.SemaphoreType.DMA,
                pltpu.SemaphoreType.DMA, pltpu.SemaphoreType.DMA,
                pltpu.SemaphoreType.DMA, pltpu.SemaphoreType.DMA,
            ],
            compiler_params=_SC_PARAMS,
        )(src, dst, ex, *h_arrs, zin)

    return run


_num_pass1 = _make_num_pass(SCHED1, 3)
_num_pass2 = _make_num_pass(SCHED2, 6)
_num_pass3 = _make_num_pass(SCHED3, 12)


def _pq_body(src_h, dst_h, p_h, q_h, sp_h, sq_h, trash_h,
             srcv0, srcv1, dstv0, dstv1, bp0, bp1, bq0, bq1,
             si0, si1, sg0, sg1, sw0, sw1):
    srcv = [srcv0, srcv1]
    dstv = [dstv0, dstv1]
    bp = [bp0, bp1]
    bq = [bq0, bq1]
    si = [si0, si1]
    sg = [sg0, sg1]
    sw = [sw0, sw1]

    cid = lax.axis_index("c")
    sid = lax.axis_index("s")
    w = sid * NC + cid
    per = NBLK // NT
    jmax = jnp.where(w < NBLK - per * NT, per, per - 1)

    def base_of(j):
        jj = jnp.minimum(j, jmax)
        return (w + jj * NT) * B

    def idx_start(j, p):
        base = base_of(j)
        pltpu.make_async_copy(src_h.at[pl.ds(base, B)], srcv[p], si[p]).start()
        pltpu.make_async_copy(dst_h.at[pl.ds(base, B)], dstv[p], si[p]).start()

    def idx_wait(p):
        pltpu.make_async_copy(src_h.at[pl.ds(0, B)], srcv[p], si[p]).wait()
        pltpu.make_async_copy(dst_h.at[pl.ds(0, B)], dstv[p], si[p]).wait()

    def gath_start(p):
        pltpu.make_async_copy(p_h.at[srcv[p]], bp[p], sg[p]).start()
        pltpu.make_async_copy(q_h.at[dstv[p]], bq[p], sg[p]).start()

    def gath_wait(p):
        pltpu.make_async_copy(p_h.at[srcv[p]], bp[p], sg[p]).wait()
        pltpu.make_async_copy(q_h.at[dstv[p]], bq[p], sg[p]).wait()

    idx_start(0, 0)
    idx_start(1, 1)
    idx_wait(0)
    gath_start(0)
    pltpu.make_async_copy(bp[0], trash_h, sw[0]).start()
    pltpu.make_async_copy(bq[0], trash_h, sw[0]).start()
    pltpu.make_async_copy(bp[1], trash_h, sw[1]).start()
    pltpu.make_async_copy(bq[1], trash_h, sw[1]).start()

    def pair(t, _):
        for p in (0, 1):
            j = 2 * t + p
            gath_wait(p)
            pltpu.make_async_copy(bp[p], trash_h, sw[p]).wait()
            pltpu.make_async_copy(bq[p], trash_h, sw[p]).wait()
            base = base_of(j)
            pltpu.make_async_copy(bp[p], sp_h.at[pl.ds(base, B)],
                                  sw[p]).start()
            pltpu.make_async_copy(bq[p], sq_h.at[pl.ds(base, B)],
                                  sw[p]).start()
            idx_start(j + 2, p)
            idx_wait(1 - p)
            gath_start(1 - p)
        return 0

    lax.fori_loop(0, NBPT // 2, pair, 0)
    gath_wait(0)
    idx_wait(1)
    for p in (0, 1):
        pltpu.make_async_copy(bp[p], trash_h, sw[p]).wait()
        pltpu.make_async_copy(bq[p], trash_h, sw[p]).wait()


def _pq_pass(src, dst, p, q):
    sp, sq, _ = pl.kernel(
        _pq_body,
        out_type=[jax.ShapeDtypeStruct((E, 64), jnp.float32),
                  jax.ShapeDtypeStruct((E, 64), jnp.float32),
                  jax.ShapeDtypeStruct((B, 64), jnp.float32)],
        mesh=_MESH,
        scratch_types=[
            pltpu.VMEM((B,), jnp.int32), pltpu.VMEM((B,), jnp.int32),
            pltpu.VMEM((B,), jnp.int32), pltpu.VMEM((B,), jnp.int32),
            pltpu.VMEM((B, 64), jnp.float32), pltpu.VMEM((B, 64), jnp.float32),
            pltpu.VMEM((B, 64), jnp.float32), pltpu.VMEM((B, 64), jnp.float32),
            pltpu.SemaphoreType.DMA, pltpu.SemaphoreType.DMA,
            pltpu.SemaphoreType.DMA, pltpu.SemaphoreType.DMA,
            pltpu.SemaphoreType.DMA, pltpu.SemaphoreType.DMA,
        ],
        compiler_params=_SC_PARAMS,
    )(src, dst, p, q)
    return sp, sq


def _xstats_body(x_ref, s_ref, q_ref):
    i = pl.program_id(0)
    z = x_ref[...]
    s = jnp.broadcast_to(jnp.sum(z, axis=0, keepdims=True), (8, 16))
    q = jnp.broadcast_to(jnp.sum(z * z, axis=0, keepdims=True), (8, 16))

    @pl.when(i == 0)
    def _():
        s_ref[...] = s
        q_ref[...] = q

    @pl.when(i > 0)
    def _():
        s_ref[...] += s
        q_ref[...] += q


def _xstats(x):
    XB = 5000
    return pl.pallas_call(
        _xstats_body,
        grid=(N // XB,),
        in_specs=[pl.BlockSpec((XB, 16), lambda i: (i, 0))],
        out_specs=[pl.BlockSpec((8, 16), lambda i: (0, 0))] * 2,
        out_shape=[jax.ShapeDtypeStruct((8, 16), jnp.float32)] * 2,
    )(x)


def _k1_body(x_ref, xs_ref, xq_ref, g_ref, b_ref, w_ref, as_ref,
             h0, h1, h2, al_ref):
    x = x_ref[...]
    m = xs_ref[0:1, :] * (1.0 / N)
    v = xq_ref[0:1, :] * (1.0 / N) - m * m
    xb = (x - m) * (g_ref[...] / jnp.sqrt(v + EPS)) + b_ref[...]
    hf = jnp.dot(xb, w_ref[...], preferred_element_type=jnp.float32)
    h0[...] = hf[:, 0:16]
    h1[...] = hf[:, 16:32]
    h2[...] = hf[:, 32:48]
    al_ref[...] = jnp.dot(hf, as_ref[...], preferred_element_type=jnp.float32)


def _k1(x, g, b, W1, as1):
    XB = 5000
    xs, xq = _xstats(x)
    full = lambda a: pl.BlockSpec(a.shape, lambda i: (0,) * a.ndim)
    args = [x, xs, xq, g.reshape(1, 16), b.reshape(1, 16), W1, as1]
    return pl.pallas_call(
        _k1_body,
        grid=(N // XB,),
        in_specs=[pl.BlockSpec((XB, 16), lambda i: (i, 0))]
        + [full(a) for a in args[1:]],
        out_specs=[pl.BlockSpec((XB, 16), lambda i: (i, 0))] * 4,
        out_shape=[jax.ShapeDtypeStruct((N, 16), jnp.float32)] * 4,
    )(*args)


def _make_k2(sched, F, n_h, FN):
    num_src, den_src, nrounds = _sched_meta(sched, F)
    nfs = F // 16
    n_rs = 2 * nrounds

    def body(rs, al_ref, h_arrs, b_ref, wn_ref, asn_ref, outs):
        rsum = [rs[2 * r][...] + rs[2 * r + 1][...] for r in range(nrounds)]
        als = al_ref[...]
        acc = None
        for h in range(H):
            exs = jnp.exp(_leaky(als[:, h:h + 1] + als[:, 3 + h:4 + h], ALEAK))
            num = jnp.concatenate(
                [rsum[num_src[(h, fs)][0]]
                 [:, num_src[(h, fs)][1]:num_src[(h, fs)][1] + 16]
                 for fs in range(nfs)], axis=1)
            hself = jnp.concatenate(
                [h_arrs[h * nfs + fs][...] for fs in range(nfs)], axis=1)
            num = num + exs * hself
            dpos, doff = den_src[h]
            den = rsum[dpos][:, doff:doff + 1] + exs
            term = num / den
            acc = term if acc is None else acc + term
        out = acc * (1.0 / H) + b_ref[...]
        hn = jnp.dot(out, wn_ref[...], preferred_element_type=jnp.float32)
        if asn_ref is not None:
            for j in range(FN // 16):
                outs[j][...] = hn[:, 16 * j:16 * j + 16]
            outs[FN // 16][...] = jnp.dot(
                hn, asn_ref[...], preferred_element_type=jnp.float32)
        else:
            outs[0][...] = hn[:, 0:64]
            outs[1][...] = hn[:, 64:128]

    RB = 1000
    grid = (N // RB,)

    def run(rs, al, h_arrs, bias, wn, asn):
        has_asn = asn is not None
        if has_asn:
            out_shape = [jax.ShapeDtypeStruct((N, 16), jnp.float32)] \
                * (FN // 16 + 1)
            out_specs = [pl.BlockSpec((RB, 16), lambda i: (i, 0))
                         for _ in range(FN // 16 + 1)]
        else:
            out_shape = [jax.ShapeDtypeStruct((N, 64), jnp.float32)] * 2
            out_specs = [pl.BlockSpec((RB, 64), lambda i: (i, 0))] * 2

        row_spec16 = pl.BlockSpec((RB, 16), lambda i: (i, 0))
        row_spec32 = pl.BlockSpec((RB, 32), lambda i: (i, 0))
        full = lambda a: pl.BlockSpec(a.shape, lambda i: (0,) * a.ndim)

        args = list(rs) + [al] + list(h_arrs) + [bias, wn]
        in_specs = ([row_spec32] * len(rs) + [row_spec16]
                    + [row_spec16] * len(h_arrs) + [full(bias), full(wn)])
        if has_asn:
            args.append(asn)
            in_specs.append(full(asn))

        def body_call(*refs):
            ins = list(refs[:len(args)])
            outs2 = refs[len(args):]
            asn_ref = ins.pop() if has_asn else None
            wn_r = ins.pop()
            b_r = ins.pop()
            body(ins[:n_rs], ins[n_rs], ins[n_rs + 1:],
                 b_r, wn_r, asn_ref, outs2)

        return pl.pallas_call(
            body_call,
            grid=grid,
            in_specs=in_specs,
            out_specs=out_specs,
            out_shape=out_shape,
        )(*args)

    return run


_k2_1 = _make_k2(SCHED1, 16, 3, 96)
_k2_2 = _make_k2(SCHED2, 32, 6, 192)
_k2_3 = _make_k2(SCHED3, 64, 12, 128)


def _estats_body(e_ref, s_ref, q_ref):
    i = pl.program_id(0)
    z = e_ref[...]
    s = jnp.broadcast_to(jnp.sum(z, axis=0, keepdims=True), (8, 10))
    q = jnp.broadcast_to(jnp.sum(z * z, axis=0, keepdims=True), (8, 10))

    @pl.when(i == 0)
    def _():
        s_ref[...] = s
        q_ref[...] = q

    @pl.when(i > 0)
    def _():
        s_ref[...] += s
        q_ref[...] += q


def _estats(e):
    EB = 8000
    return pl.pallas_call(
        _estats_body,
        grid=(E // EB,),
        in_specs=[pl.BlockSpec((EB, 10), lambda i: (i, 0))],
        out_specs=[pl.BlockSpec((8, 10), lambda i: (0, 0))] * 2,
        out_shape=[jax.ShapeDtypeStruct((8, 10), jnp.float32)] * 2,
    )(e)


def _mlp_body(sp, sq, e_ref, es_ref, eq_ref, g_ref, bb_ref, c_ref, be1,
              w2, b2, w3, b3, w4, b4, w5, b5, o_ref):
    m = es_ref[0:1, :] * (1.0 / E)
    v = eq_ref[0:1, :] * (1.0 / E) - m * m
    sc = g_ref[...] / jnp.sqrt(v + EPS)
    eb = (e_ref[...] - m) * sc + bb_ref[...]
    z = sp[...] + sq[...] + jnp.dot(
        eb, c_ref[...], preferred_element_type=jnp.float32) + be1[...]
    z = _leaky(z, LEAK)
    z = _leaky(jnp.dot(z, w2[...], preferred_element_type=jnp.float32)
               + b2[...], LEAK)
    z = _leaky(jnp.dot(z, w3[...], preferred_element_type=jnp.float32)
               + b3[...], LEAK)
    z = _leaky(jnp.dot(z, w4[...], preferred_element_type=jnp.float32)
               + b4[...], LEAK)
    o_ref[...] = jnp.dot(z, w5[...], preferred_element_type=jnp.float32) \
        + b5[...]


def _mlp(sp, sq, e, es, eq, g, bb, C, be1, w2, b2, w3, b3, w4, b4, w5, b5):
    EB = 4000
    full = lambda a: pl.BlockSpec(a.shape, lambda i: (0,) * a.ndim)
    row = lambda w: pl.BlockSpec((EB, w), lambda i: (i, 0))
    args = [sp, sq, e, es, eq, g, bb, C, be1, w2, b2, w3, b3, w4, b4, w5, b5]
    in_specs = [row(64), row(64), row(10)] + [full(a) for a in args[3:]]
    return pl.pallas_call(
        _mlp_body,
        grid=(E // EB,),
        in_specs=in_specs,
        out_specs=pl.BlockSpec((EB, 2), lambda i: (i, 0)),
        out_shape=jax.ShapeDtypeStruct((E, 2), jnp.float32),
    )(*args)


def _as_mat(a_s, a_d, F):
    M = jnp.zeros((H * F, 16), jnp.float32)
    rows = jnp.arange(H * F)
    heads = jnp.repeat(jnp.arange(H), F)
    M = M.at[rows, heads].set(a_s.reshape(-1))
    M = M.at[rows, 3 + heads].set(a_d.reshape(-1))
    return M


def kernel(x, edge_index, e, xbatch, bn_node_g, bn_node_b, bn_edge_g,
           bn_edge_b, W1, a_s1, a_d1, b1, W2, a_s2, a_d2, b2, W3, a_s3,
           a_d3, b3, We1, be1, We2, be2, We3, be3, We4, be4, We5, be5):
    src = edge_index[0]
    dst = edge_index[1]
    zin = jnp.zeros((ZR, 32), jnp.float32)

    as1 = _as_mat(a_s1, a_d1, 16)
    as2 = _as_mat(a_s2, a_d2, 32)
    as3 = _as_mat(a_s3, a_d3, 64)

    *h1_arrs, al1 = _k1(x, bn_node_g, bn_node_b, W1, as1)
    ex1 = _ex_pass(src, dst, al1)
    r1 = _num_pass1(src, dst, ex1, h1_arrs, zin)
    out2 = _k2_1(r1, al1, h1_arrs, b1.reshape(1, 16), W2, as2)
    h2_arrs, al2 = out2[:6], out2[6]

    ex2 = _ex_pass(src, dst, al2)
    r2 = _num_pass2(src, dst, ex2, h2_arrs, zin)
    out3 = _k2_2(r2, al2, h2_arrs, b2.reshape(1, 32), W3, as3)
    h3_arrs, al3 = out3[:12], out3[12]

    ex3 = _ex_pass(src, dst, al3)
    r3 = _num_pass3(src, dst, ex3, h3_arrs, zin)
    pq_w = jnp.concatenate([We1[0:64], We1[64:128]], axis=1)
    p, q = _k2_3(r3, al3, h3_arrs, b3.reshape(1, 64), pq_w, None)

    sp, sq = _pq_pass(src, dst, p, q)
    es, eq = _estats(e)
    return _mlp(sp, sq, e, es, eq, bn_edge_g.reshape(1, 10),
                bn_edge_b.reshape(1, 10), We1[128:138], be1.reshape(1, 64),
                We2, be2.reshape(1, 32), We3, be3.reshape(1, 16),
                We4, be4.reshape(1, 8), We5, be5.reshape(1, 2))


# speedup vs baseline: 20.3845x; 1.0291x over previous
import jax
import jax.numpy as jnp
from jax import lax
from jax.experimental import pallas as pl
from jax.experimental.pallas import tpu as pltpu
from jax.experimental.pallas import tpu_sc as plsc

N = 50000
E = 800000
H = 3
LEAK = 0.1
ALEAK = 0.2
EPS = 1e-5

B = 128
NBLK = E // B
NC, NS, NT = 2, 16, 32
RPT = N // NS
ZR = 125
HALF = NBLK // 2
NBPT = 196

_MESH = plsc.VectorSubcoreMesh(core_axis_name="c", subcore_axis_name="s",
                               num_cores=NC, num_subcores=NS)
_SC_PARAMS = pltpu.CompilerParams(needs_layout_passes=False,
                                  use_tc_tiling_on_sc=False)


def _s16(v):
    return jnp.full((16,), v, jnp.int32)


def _leaky(x, slope):
    return jnp.maximum(x, slope * x)


def _copy128(src, dst):
    for q in range(8):
        dst[pl.ds(q * 16, 16)] = src[pl.ds(q * 16, 16)]


def _ex_body(src_h, dst_h, al_h, ex_h, trash_h,
             srcv0, srcv1, dstv0, dstv1, sv0, sv1, dv0, dv1, exv0, exv1,
             si0, si1, sg0, sg1, sw0, sw1):
    srcv = [srcv0, srcv1]
    dstv = [dstv0, dstv1]
    sv = [sv0, sv1]
    dv = [dv0, dv1]
    exv = [exv0, exv1]
    si = [si0, si1]
    sg = [sg0, sg1]
    sw = [sw0, sw1]

    cid = lax.axis_index("c")
    sid = lax.axis_index("s")
    w = sid * NC + cid
    iota = lax.iota(jnp.int32, 16)
    per = NBLK // NT
    jmax = jnp.where(w < NBLK - per * NT, per, per - 1)

    def base_of(j):
        jj = jnp.minimum(j, jmax)
        return (w + jj * NT) * B

    def idx_start(j, p):
        base = base_of(j)
        pltpu.make_async_copy(src_h.at[pl.ds(base, B)], srcv[p], si[p]).start()
        pltpu.make_async_copy(dst_h.at[pl.ds(base, B)], dstv[p], si[p]).start()

    def idx_wait(p):
        pltpu.make_async_copy(src_h.at[pl.ds(0, B)], srcv[p], si[p]).wait()
        pltpu.make_async_copy(dst_h.at[pl.ds(0, B)], dstv[p], si[p]).wait()

    def gath_start(p):
        pltpu.make_async_copy(al_h.at[srcv[p]], sv[p], sg[p]).start()
        pltpu.make_async_copy(al_h.at[dstv[p]], dv[p], sg[p]).start()

    def gath_wait(p):
        pltpu.make_async_copy(al_h.at[srcv[p]], sv[p], sg[p]).wait()
        pltpu.make_async_copy(al_h.at[dstv[p]], dv[p], sg[p]).wait()

    idx_start(0, 0)
    idx_start(1, 1)
    idx_wait(0)
    gath_start(0)

    def pair(t, _):
        for p in (0, 1):
            j = 2 * t + p
            idx_wait(1 - p)
            gath_start(1 - p)
            gath_wait(p)

            @pl.when(t > 0)
            def _():
                pltpu.make_async_copy(exv[p], trash_h, sw[p]).wait()

            def grp_body(g, _):
                rows = g * 16 + iota
                for h in range(H):
                    a = plsc.load_gather(sv[p], [rows, _s16(h)])
                    b = plsc.load_gather(dv[p], [rows, _s16(3 + h)])
                    lo = a + b
                    plsc.store_scatter(exv[p], [rows, _s16(h)],
                                       jnp.exp(_leaky(lo, ALEAK)))
                return 0

            lax.fori_loop(0, B // 16, grp_body, 0)
            base = base_of(j)
            pltpu.make_async_copy(exv[p], ex_h.at[pl.ds(base, B)],
                                  sw[p]).start()
            idx_start(j + 2, p)
        return 0

    lax.fori_loop(0, NBPT // 2, pair, 0)
    gath_wait(0)
    idx_wait(1)
    pltpu.make_async_copy(exv[0], trash_h, sw[0]).wait()
    pltpu.make_async_copy(exv[1], trash_h, sw[1]).wait()


def _ex_pass(src, dst, al):
    ex, _ = pl.kernel(
        _ex_body,
        out_type=[jax.ShapeDtypeStruct((E, 4), jnp.float32),
                  jax.ShapeDtypeStruct((B, 4), jnp.float32)],
        mesh=_MESH,
        scratch_types=[
            pltpu.VMEM((B,), jnp.int32), pltpu.VMEM((B,), jnp.int32),
            pltpu.VMEM((B,), jnp.int32), pltpu.VMEM((B,), jnp.int32),
            pltpu.VMEM((B, 16), jnp.float32), pltpu.VMEM((B, 16), jnp.float32),
            pltpu.VMEM((B, 16), jnp.float32), pltpu.VMEM((B, 16), jnp.float32),
            pltpu.VMEM((B, 4), jnp.float32), pltpu.VMEM((B, 4), jnp.float32),
            pltpu.SemaphoreType.DMA, pltpu.SemaphoreType.DMA,
            pltpu.SemaphoreType.DMA, pltpu.SemaphoreType.DMA,
            pltpu.SemaphoreType.DMA, pltpu.SemaphoreType.DMA,
        ],
        compiler_params=_SC_PARAMS,
    )(src, dst, al)
    return ex


SCHED1 = [[("num", 0, 0), ("den", 0)],
          [("num", 1, 1), ("den", 1)],
          [("num", 2, 2), ("den", 2)]]
SCHED2 = [[("num", 0, 0), ("num", 0, 1)],
          [("num", 1, 2), ("num", 1, 3)],
          [("num", 2, 4), ("num", 2, 5)],
          [("den", 0), ("den", 1)],
          [("den", 2)]]
SCHED3 = [[("num", 0, 0), ("num", 0, 1)],
          [("num", 0, 2), ("num", 0, 3)],
          [("num", 1, 4), ("num", 1, 5)],
          [("num", 1, 6), ("num", 1, 7)],
          [("num", 2, 8), ("num", 2, 9)],
          [("num", 2, 10), ("num", 2, 11)],
          [("den", 0), ("den", 1)],
          [("den", 2)]]


def _sched_meta(sched, F):
    num_src = {}
    den_src = {}
    for r, chunks in enumerate(sched):
        for k, ch in enumerate(chunks):
            if ch[0] == "num":
                fs = ch[2] - ch[1] * (F // 16)
                num_src[(ch[1], fs)] = (r, 16 * k)
            else:
                den_src[ch[1]] = (r, 16 * k)
    return num_src, den_src, len(sched)


def _make_num_pass(sched, n_h):
    nrounds = len(sched)
    n_out = 2 * nrounds

    def body(*refs):
        (src_h, dst_h, ex_h), refs = refs[:3], refs[3:]
        h_arrs, refs = refs[:n_h], refs[n_h:]
        (zin_h,), refs = refs[:1], refs[1:]
        outs, refs = refs[:n_out], refs[n_out:]
        (srcv0, srcv1, dstv0, dstv1, dsc0, dsc1, exv0, exv1,
         ga0, ga1, gb0, gb1, scl0, scl1, zbuf, T,
         si0, si1, sg0, sg1, ss0, ss1) = refs
        srcv = [srcv0, srcv1]
        dstv = [dstv0, dstv1]
        dsc = [dsc0, dsc1]
        exv = [exv0, exv1]
        gk = [[ga0, gb0], [ga1, gb1]]
        scl = [scl0, scl1]
        si = [si0, si1]
        sg = [sg0, sg1]
        ss = [ss0, ss1]

        cid = lax.axis_index("c")
        sid = lax.axis_index("s")
        iota = lax.iota(jnp.int32, 16)
        chbase = cid * HALF
        per = HALF // NS
        jmax = jnp.where(sid < HALF - per * NS, per, per - 1)

        pltpu.sync_copy(zin_h, zbuf)

        for r, chunks in enumerate(sched):
            numk = [k for k, ch in enumerate(chunks) if ch[0] == "num"]

            def base_of(j):
                jj = jnp.minimum(j, jmax)
                return (chbase + sid + jj * NS) * B

            def idx_start(j, p):
                base = base_of(j)
                pltpu.make_async_copy(
                    dst_h.at[pl.ds(base, B)], dstv[p], si[p]).start()
                pltpu.make_async_copy(
                    ex_h.at[pl.ds(base, B)], exv[p], si[p]).start()
                if numk:
                    pltpu.make_async_copy(
                        src_h.at[pl.ds(base, B)], srcv[p], si[p]).start()

            def idx_wait(p):
                pltpu.make_async_copy(
                    dst_h.at[pl.ds(0, B)], dstv[p], si[p]).wait()
                pltpu.make_async_copy(
                    ex_h.at[pl.ds(0, B)], exv[p], si[p]).wait()
                if numk:
                    pltpu.make_async_copy(
                        src_h.at[pl.ds(0, B)], srcv[p], si[p]).wait()

            def gath_start(p):
                for k in numk:
                    pltpu.make_async_copy(
                        h_arrs[chunks[k][2]].at[srcv[p]], gk[p][k],
                        sg[p]).start()

            def gath_wait(p):
                for k in numk:
                    pltpu.make_async_copy(
                        h_arrs[chunks[k][2]].at[srcv[p]], gk[p][k],
                        sg[p]).wait()

            def scat_start(p):
                pltpu.make_async_copy(
                    scl[p], T.at[dsc[p]], ss[p]).start(add=True)

            def scat_wait(p):
                pltpu.make_async_copy(scl[p], T.at[dsc[p]], ss[p]).wait()

            def zb(jz, _):
                pltpu.sync_copy(zbuf, T.at[pl.ds(sid * RPT + jz * ZR, ZR)])
                return 0

            lax.fori_loop(0, RPT // ZR, zb, 0)
            plsc.subcore_barrier()

            idx_start(0, 0)
            idx_start(1, 1)
            idx_wait(0)
            gath_start(0)

            def pair(t, _):
                for p in (0, 1):
                    j = 2 * t + p
                    idx_wait(1 - p)
                    gath_start(1 - p)
                    gath_wait(p)

                    @pl.when(t > 0)
                    def _():
                        scat_wait(p)

                    vf = jnp.where(
                        (sid + j * NS) < HALF, 1.0, 0.0).astype(jnp.float32)

                    def grp_body(g, _):
                        rows = g * 16 + iota
                        for k, ch in enumerate(chunks):
                            exg = plsc.load_gather(
                                exv[p], [rows, _s16(ch[1])]) * vf
                            if ch[0] == "num":
                                def fb(fi, _, k=k, exg=exg, rows=rows):
                                    for f2 in range(8):
                                        f = fi * 8 + f2
                                        col = plsc.load_gather(
                                            gk[p][k], [rows, _s16(f)])
                                        plsc.store_scatter(
                                            scl[p], [rows, _s16(16 * k + f)],
                                            col * exg)
                                    return 0
                                lax.fori_loop(0, 2, fb, 0)
                            else:
                                def fb(fi, _, k=k, exg=exg, rows=rows):
                                    for f2 in range(8):
                                        f = fi * 8 + f2
                                        plsc.store_scatter(
                                            scl[p], [rows, _s16(16 * k + f)],
                                            exg)
                                    return 0
                                lax.fori_loop(0, 2, fb, 0)
                        return 0

                    lax.fori_loop(0, B // 16, grp_body, 0)
                    _copy128(dstv[p], dsc[p])
                    scat_start(p)
                    idx_start(j + 2, p)
                return 0

            lax.fori_loop(0, NBPT // 2, pair, 0)
            gath_wait(0)
            idx_wait(1)
            scat_wait(0)
            scat_wait(1)
            plsc.subcore_barrier()

            @pl.when(cid == 0)
            def _(r=r):
                pltpu.sync_copy(T.at[pl.ds(sid * RPT, RPT)],
                                outs[2 * r].at[pl.ds(sid * RPT, RPT)])

            @pl.when(cid == 1)
            def _(r=r):
                pltpu.sync_copy(T.at[pl.ds(sid * RPT, RPT)],
                                outs[2 * r + 1].at[pl.ds(sid * RPT, RPT)])

    def run(src, dst, ex, h_arrs, zin):
        return pl.kernel(
            body,
            out_type=[jax.ShapeDtypeStruct((N, 32), jnp.float32)] * n_out,
            mesh=_MESH,
            scratch_types=[
                pltpu.VMEM((B,), jnp.int32), pltpu.VMEM((B,), jnp.int32),
                pltpu.VMEM((B,), jnp.int32), pltpu.VMEM((B,), jnp.int32),
                pltpu.VMEM((B,), jnp.int32), pltpu.VMEM((B,), jnp.int32),
                pltpu.VMEM((B, 4), jnp.float32),
                pltpu.VMEM((B, 4), jnp.float32),
                pltpu.VMEM((B, 16), jnp.float32),
                pltpu.VMEM((B, 16), jnp.float32),
                pltpu.VMEM((B, 16), jnp.float32),
                pltpu.VMEM((B, 16), jnp.float32),
                pltpu.VMEM((B, 32), jnp.float32),
                pltpu.VMEM((B, 32), jnp.float32),
                pltpu.VMEM((ZR, 32), jnp.float32),
                pltpu.VMEM_SHARED((N, 32), jnp.float32),
                pltpu.SemaphoreType.DMA, pltpu.SemaphoreType.DMA,
                pltpu.SemaphoreType.DMA, pltpu.SemaphoreType.DMA,
                pltpu.SemaphoreType.DMA, pltpu.SemaphoreType.DMA,
            ],
            compiler_params=_SC_PARAMS,
        )(src, dst, ex, *h_arrs, zin)

    return run


_num_pass1 = _make_num_pass(SCHED1, 3)
_num_pass2 = _make_num_pass(SCHED2, 6)
_num_pass3 = _make_num_pass(SCHED3, 12)


def _pq_body(src_h, dst_h, p_h, q_h, sp_h, sq_h, trash_h,
             srcv0, srcv1, dstv0, dstv1, bp0, bp1, bq0, bq1,
             si0, si1, sg0, sg1, sw0, sw1):
    srcv = [srcv0, srcv1]
    dstv = [dstv0, dstv1]
    bp = [bp0, bp1]
    bq = [bq0, bq1]
    si = [si0, si1]
    sg = [sg0, sg1]
    sw = [sw0, sw1]

    cid = lax.axis_index("c")
    sid = lax.axis_index("s")
    w = sid * NC + cid
    per = NBLK // NT
    jmax = jnp.where(w < NBLK - per * NT, per, per - 1)

    def base_of(j):
        jj = jnp.minimum(j, jmax)
        return (w + jj * NT) * B

    def idx_start(j, p):
        base = base_of(j)
        pltpu.make_async_copy(src_h.at[pl.ds(base, B)], srcv[p], si[p]).start()
        pltpu.make_async_copy(dst_h.at[pl.ds(base, B)], dstv[p], si[p]).start()

    def idx_wait(p):
        pltpu.make_async_copy(src_h.at[pl.ds(0, B)], srcv[p], si[p]).wait()
        pltpu.make_async_copy(dst_h.at[pl.ds(0, B)], dstv[p], si[p]).wait()

    def gath_start(p):
        pltpu.make_async_copy(p_h.at[srcv[p]], bp[p], sg[p]).start()
        pltpu.make_async_copy(q_h.at[dstv[p]], bq[p], sg[p]).start()

    def gath_wait(p):
        pltpu.make_async_copy(p_h.at[srcv[p]], bp[p], sg[p]).wait()
        pltpu.make_async_copy(q_h.at[dstv[p]], bq[p], sg[p]).wait()

    idx_start(0, 0)
    idx_start(1, 1)
    idx_wait(0)
    gath_start(0)

    def pair(t, _):
        for p in (0, 1):
            j = 2 * t + p
            idx_wait(1 - p)

            if p == 0:
                @pl.when(t > 0)
                def _():
                    pltpu.make_async_copy(bp[1], trash_h, sw[1]).wait()
                    pltpu.make_async_copy(bq[1], trash_h, sw[1]).wait()
            else:
                pltpu.make_async_copy(bp[0], trash_h, sw[0]).wait()
                pltpu.make_async_copy(bq[0], trash_h, sw[0]).wait()

            gath_start(1 - p)
            gath_wait(p)
            base = base_of(j)
            pltpu.make_async_copy(bp[p], sp_h.at[pl.ds(base, B)],
                                  sw[p]).start()
            pltpu.make_async_copy(bq[p], sq_h.at[pl.ds(base, B)],
                                  sw[p]).start()
            idx_start(j + 2, p)
        return 0

    lax.fori_loop(0, NBPT // 2, pair, 0)
    gath_wait(0)
    idx_wait(1)
    pltpu.make_async_copy(bp[1], trash_h, sw[1]).wait()
    pltpu.make_async_copy(bq[1], trash_h, sw[1]).wait()


def _pq_pass(src, dst, p, q):
    sp, sq, _ = pl.kernel(
        _pq_body,
        out_type=[jax.ShapeDtypeStruct((E, 64), jnp.float32),
                  jax.ShapeDtypeStruct((E, 64), jnp.float32),
                  jax.ShapeDtypeStruct((B, 64), jnp.float32)],
        mesh=_MESH,
        scratch_types=[
            pltpu.VMEM((B,), jnp.int32), pltpu.VMEM((B,), jnp.int32),
            pltpu.VMEM((B,), jnp.int32), pltpu.VMEM((B,), jnp.int32),
            pltpu.VMEM((B, 64), jnp.float32), pltpu.VMEM((B, 64), jnp.float32),
            pltpu.VMEM((B, 64), jnp.float32), pltpu.VMEM((B, 64), jnp.float32),
            pltpu.SemaphoreType.DMA, pltpu.SemaphoreType.DMA,
            pltpu.SemaphoreType.DMA, pltpu.SemaphoreType.DMA,
            pltpu.SemaphoreType.DMA, pltpu.SemaphoreType.DMA,
        ],
        compiler_params=_SC_PARAMS,
    )(src, dst, p, q)
    return sp, sq


def _xstats_body(x_ref, s_ref, q_ref):
    i = pl.program_id(0)
    z = x_ref[...]
    s = jnp.broadcast_to(jnp.sum(z, axis=0, keepdims=True), (8, 16))
    q = jnp.broadcast_to(jnp.sum(z * z, axis=0, keepdims=True), (8, 16))

    @pl.when(i == 0)
    def _():
        s_ref[...] = s
        q_ref[...] = q

    @pl.when(i > 0)
    def _():
        s_ref[...] += s
        q_ref[...] += q


def _xstats(x):
    XB = 5000
    return pl.pallas_call(
        _xstats_body,
        grid=(N // XB,),
        in_specs=[pl.BlockSpec((XB, 16), lambda i: (i, 0))],
        out_specs=[pl.BlockSpec((8, 16), lambda i: (0, 0))] * 2,
        out_shape=[jax.ShapeDtypeStruct((8, 16), jnp.float32)] * 2,
    )(x)


def _k1_body(x_ref, xs_ref, xq_ref, g_ref, b_ref, w_ref, as_ref,
             h0, h1, h2, al_ref):
    x = x_ref[...]
    m = xs_ref[0:1, :] * (1.0 / N)
    v = xq_ref[0:1, :] * (1.0 / N) - m * m
    xb = (x - m) * (g_ref[...] / jnp.sqrt(v + EPS)) + b_ref[...]
    hf = jnp.dot(xb, w_ref[...], preferred_element_type=jnp.float32)
    h0[...] = hf[:, 0:16]
    h1[...] = hf[:, 16:32]
    h2[...] = hf[:, 32:48]
    al_ref[...] = jnp.dot(hf, as_ref[...], preferred_element_type=jnp.float32)


def _k1(x, g, b, W1, as1):
    XB = 5000
    xs, xq = _xstats(x)
    full = lambda a: pl.BlockSpec(a.shape, lambda i: (0,) * a.ndim)
    args = [x, xs, xq, g.reshape(1, 16), b.reshape(1, 16), W1, as1]
    return pl.pallas_call(
        _k1_body,
        grid=(N // XB,),
        in_specs=[pl.BlockSpec((XB, 16), lambda i: (i, 0))]
        + [full(a) for a in args[1:]],
        out_specs=[pl.BlockSpec((XB, 16), lambda i: (i, 0))] * 4,
        out_shape=[jax.ShapeDtypeStruct((N, 16), jnp.float32)] * 4,
    )(*args)


def _make_k2(sched, F, n_h, FN):
    num_src, den_src, nrounds = _sched_meta(sched, F)
    nfs = F // 16
    n_rs = 2 * nrounds

    def body(rs, al_ref, h_arrs, b_ref, wn_ref, asn_ref, outs):
        rsum = [rs[2 * r][...] + rs[2 * r + 1][...] for r in range(nrounds)]
        als = al_ref[...]
        acc = None
        for h in range(H):
            exs = jnp.exp(_leaky(als[:, h:h + 1] + als[:, 3 + h:4 + h], ALEAK))
            num = jnp.concatenate(
                [rsum[num_src[(h, fs)][0]]
                 [:, num_src[(h, fs)][1]:num_src[(h, fs)][1] + 16]
                 for fs in range(nfs)], axis=1)
            hself = jnp.concatenate(
                [h_arrs[h * nfs + fs][...] for fs in range(nfs)], axis=1)
            num = num + exs * hself
            dpos, doff = den_src[h]
            den = rsum[dpos][:, doff:doff + 1] + exs
            term = num / den
            acc = term if acc is None else acc + term
        out = acc * (1.0 / H) + b_ref[...]
        hn = jnp.dot(out, wn_ref[...], preferred_element_type=jnp.float32)
        if asn_ref is not None:
            for j in range(FN // 16):
                outs[j][...] = hn[:, 16 * j:16 * j + 16]
            outs[FN // 16][...] = jnp.dot(
                hn, asn_ref[...], preferred_element_type=jnp.float32)
        else:
            outs[0][...] = hn[:, 0:64]
            outs[1][...] = hn[:, 64:128]

    RB = 1000
    grid = (N // RB,)

    def run(rs, al, h_arrs, bias, wn, asn):
        has_asn = asn is not None
        if has_asn:
            out_shape = [jax.ShapeDtypeStruct((N, 16), jnp.float32)] \
                * (FN // 16 + 1)
            out_specs = [pl.BlockSpec((RB, 16), lambda i: (i, 0))
                         for _ in range(FN // 16 + 1)]
        else:
            out_shape = [jax.ShapeDtypeStruct((N, 64), jnp.float32)] * 2
            out_specs = [pl.BlockSpec((RB, 64), lambda i: (i, 0))] * 2

        row_spec16 = pl.BlockSpec((RB, 16), lambda i: (i, 0))
        row_spec32 = pl.BlockSpec((RB, 32), lambda i: (i, 0))
        full = lambda a: pl.BlockSpec(a.shape, lambda i: (0,) * a.ndim)

        args = list(rs) + [al] + list(h_arrs) + [bias, wn]
        in_specs = ([row_spec32] * len(rs) + [row_spec16]
                    + [row_spec16] * len(h_arrs) + [full(bias), full(wn)])
        if has_asn:
            args.append(asn)
            in_specs.append(full(asn))

        def body_call(*refs):
            ins = list(refs[:len(args)])
            outs2 = refs[len(args):]
            asn_ref = ins.pop() if has_asn else None
            wn_r = ins.pop()
            b_r = ins.pop()
            body(ins[:n_rs], ins[n_rs], ins[n_rs + 1:],
                 b_r, wn_r, asn_ref, outs2)

        return pl.pallas_call(
            body_call,
            grid=grid,
            in_specs=in_specs,
            out_specs=out_specs,
            out_shape=out_shape,
        )(*args)

    return run


_k2_1 = _make_k2(SCHED1, 16, 3, 96)
_k2_2 = _make_k2(SCHED2, 32, 6, 192)
_k2_3 = _make_k2(SCHED3, 64, 12, 128)


def _estats_body(e_ref, s_ref, q_ref):
    i = pl.program_id(0)
    z = e_ref[...]
    s = jnp.broadcast_to(jnp.sum(z, axis=0, keepdims=True), (8, 10))
    q = jnp.broadcast_to(jnp.sum(z * z, axis=0, keepdims=True), (8, 10))

    @pl.when(i == 0)
    def _():
        s_ref[...] = s
        q_ref[...] = q

    @pl.when(i > 0)
    def _():
        s_ref[...] += s
        q_ref[...] += q


def _estats(e):
    EB = 8000
    return pl.pallas_call(
        _estats_body,
        grid=(E // EB,),
        in_specs=[pl.BlockSpec((EB, 10), lambda i: (i, 0))],
        out_specs=[pl.BlockSpec((8, 10), lambda i: (0, 0))] * 2,
        out_shape=[jax.ShapeDtypeStruct((8, 10), jnp.float32)] * 2,
    )(e)


def _mlp_body(sp, sq, e_ref, es_ref, eq_ref, g_ref, bb_ref, c_ref, be1,
              w2, b2, w3, b3, w4, b4, w5, b5, o_ref):
    m = es_ref[0:1, :] * (1.0 / E)
    v = eq_ref[0:1, :] * (1.0 / E) - m * m
    sc = g_ref[...] / jnp.sqrt(v + EPS)
    eb = (e_ref[...] - m) * sc + bb_ref[...]
    z = sp[...] + sq[...] + jnp.dot(
        eb, c_ref[...], preferred_element_type=jnp.float32) + be1[...]
    z = _leaky(z, LEAK)
    z = _leaky(jnp.dot(z, w2[...], preferred_element_type=jnp.float32)
               + b2[...], LEAK)
    z = _leaky(jnp.dot(z, w3[...], preferred_element_type=jnp.float32)
               + b3[...], LEAK)
    z = _leaky(jnp.dot(z, w4[...], preferred_element_type=jnp.float32)
               + b4[...], LEAK)
    o_ref[...] = jnp.dot(z, w5[...], preferred_element_type=jnp.float32) \
        + b5[...]


def _mlp(sp, sq, e, es, eq, g, bb, C, be1, w2, b2, w3, b3, w4, b4, w5, b5):
    EB = 4000
    full = lambda a: pl.BlockSpec(a.shape, lambda i: (0,) * a.ndim)
    row = lambda w: pl.BlockSpec((EB, w), lambda i: (i, 0))
    args = [sp, sq, e, es, eq, g, bb, C, be1, w2, b2, w3, b3, w4, b4, w5, b5]
    in_specs = [row(64), row(64), row(10)] + [full(a) for a in args[3:]]
    return pl.pallas_call(
        _mlp_body,
        grid=(E // EB,),
        in_specs=in_specs,
        out_specs=pl.BlockSpec((EB, 2), lambda i: (i, 0)),
        out_shape=jax.ShapeDtypeStruct((E, 2), jnp.float32),
    )(*args)


def _as_mat(a_s, a_d, F):
    M = jnp.zeros((H * F, 16), jnp.float32)
    rows = jnp.arange(H * F)
    heads = jnp.repeat(jnp.arange(H), F)
    M = M.at[rows, heads].set(a_s.reshape(-1))
    M = M.at[rows, 3 + heads].set(a_d.reshape(-1))
    return M


def kernel(x, edge_index, e, xbatch, bn_node_g, bn_node_b, bn_edge_g,
           bn_edge_b, W1, a_s1, a_d1, b1, W2, a_s2, a_d2, b2, W3, a_s3,
           a_d3, b3, We1, be1, We2, be2, We3, be3, We4, be4, We5, be5):
    src = edge_index[0]
    dst = edge_index[1]
    zin = jnp.zeros((ZR, 32), jnp.float32)

    as1 = _as_mat(a_s1, a_d1, 16)
    as2 = _as_mat(a_s2, a_d2, 32)
    as3 = _as_mat(a_s3, a_d3, 64)

    *h1_arrs, al1 = _k1(x, bn_node_g, bn_node_b, W1, as1)
    ex1 = _ex_pass(src, dst, al1)
    r1 = _num_pass1(src, dst, ex1, h1_arrs, zin)
    out2 = _k2_1(r1, al1, h1_arrs, b1.reshape(1, 16), W2, as2)
    h2_arrs, al2 = out2[:6], out2[6]

    ex2 = _ex_pass(src, dst, al2)
    r2 = _num_pass2(src, dst, ex2, h2_arrs, zin)
    out3 = _k2_2(r2, al2, h2_arrs, b2.reshape(1, 32), W3, as3)
    h3_arrs, al3 = out3[:12], out3[12]

    ex3 = _ex_pass(src, dst, al3)
    r3 = _num_pass3(src, dst, ex3, h3_arrs, zin)
    pq_w = jnp.concatenate([We1[0:64], We1[64:128]], axis=1)
    p, q = _k2_3(r3, al3, h3_arrs, b3.reshape(1, 64), pq_w, None)

    sp, sq = _pq_pass(src, dst, p, q)
    es, eq = _estats(e)
    return _mlp(sp, sq, e, es, eq, bn_edge_g.reshape(1, 10),
                bn_edge_b.reshape(1, 10), We1[128:138], be1.reshape(1, 64),
                We2, be2.reshape(1, 32), We3, be3.reshape(1, 16),
                We4, be4.reshape(1, 8), We5, be5.reshape(1, 2))
